# Initial kernel scaffold; baseline (speedup 1.0000x reference)
import jax
import jax.numpy as jnp
from jax.experimental import pallas as pl


def kernel(inputs, token_table, pos_table):
    raise NotImplementedError("write your pallas kernel here")


# speedup vs baseline: 1.8854x; 1.8854x over previous
import functools

import jax
import jax.numpy as jnp
from jax import lax
from jax.experimental import pallas as pl
from jax.experimental.pallas import tpu as pltpu
from jax.experimental.pallas import tpu_sc as plsc


def _build_sc_kernel(N, V, D, S):
    info = plsc.get_sparse_core_info()
    NC, NS, L = info.num_cores, info.num_subcores, info.num_lanes
    NW = NC * NS
    RPW = N // NW
    CH = 128
    NCH = RPW // CH
    assert N % NW == 0 and RPW % CH == 0 and RPW % S == 0 and D % L == 0
    assert S > CH
    PEXT = CH
    PROWS = S + PEXT

    mesh = plsc.VectorSubcoreMesh(core_axis_name="c", subcore_axis_name="s")

    @functools.partial(
        pl.kernel,
        mesh=mesh,
        out_type=jax.ShapeDtypeStruct((N, D), jnp.float32),
        scratch_types=[
            pltpu.VMEM((RPW,), jnp.int32),
            pltpu.VMEM((PROWS, D), jnp.float32),
            pltpu.VMEM((2, CH, D), jnp.float32),
            pltpu.SemaphoreType.DMA,
        ],
    )
    def k(idx_hbm, tab_hbm, pos_hbm, out_hbm, idxv, posv, gbuf, sem):
        cid = lax.axis_index("c")
        sid = lax.axis_index("s")
        w = sid * NC + cid
        base = w * RPW
        pltpu.sync_copy(idx_hbm.at[pl.ds(base, RPW)], idxv)
        pltpu.sync_copy(pos_hbm, posv.at[pl.ds(0, S)])
        pltpu.sync_copy(pos_hbm.at[pl.ds(0, PEXT)], posv.at[pl.ds(S, PEXT)])

        @pl.loop(0, NCH)
        def _(g):
            pltpu.async_copy(
                tab_hbm.at[idxv.at[pl.ds(g * CH, CH)]], gbuf.at[0], sem
            ).wait()
            p0 = lax.rem(g * CH, S)

            @pl.loop(0, CH)
            def _(r):
                pr = p0 + r
                for c in range(D // L):
                    sl = pl.ds(c * L, L)
                    gbuf[0, r, sl] = gbuf[0, r, sl] + posv[pr, sl]

            pltpu.sync_copy(gbuf.at[0], out_hbm.at[pl.ds(base + g * CH, CH)])

    return k


def kernel(inputs, token_table, pos_table):
    B, S = inputs.shape
    V, D = token_table.shape
    N = B * S
    idx = inputs.reshape(N).astype(jnp.int32)
    run = _build_sc_kernel(N, V, D, S)
    out = run(idx, token_table, pos_table)
    return out.reshape(B, S, D)


# speedup vs baseline: 2.5598x; 1.3577x over previous
import functools

import jax
import jax.numpy as jnp
from jax import lax
from jax.experimental import pallas as pl
from jax.experimental.pallas import tpu as pltpu
from jax.experimental.pallas import tpu_sc as plsc


def _build_sc_kernel(N, V, D, S):
    info = plsc.get_sparse_core_info()
    NC, NS, L = info.num_cores, info.num_subcores, info.num_lanes
    NW = NC * NS
    RPW = N // NW
    CH = 64
    NCH = RPW // CH
    NBUF = 4
    LOOK = 2
    assert N % NW == 0 and RPW % CH == 0 and RPW % S == 0 and D % L == 0
    assert S > CH and CH % 8 == 0 and NCH % NBUF == 0
    PEXT = CH
    PROWS = S + PEXT

    mesh = plsc.VectorSubcoreMesh(core_axis_name="c", subcore_axis_name="s")

    @functools.partial(
        pl.kernel,
        mesh=mesh,
        out_type=jax.ShapeDtypeStruct((N, D), jnp.float32),
        scratch_types=[
            pltpu.VMEM((RPW,), jnp.int32),
            pltpu.VMEM((PROWS, D), jnp.float32),
            pltpu.VMEM((NBUF, CH, D), jnp.float32),
            pltpu.SemaphoreType.DMA((NBUF,)),
            pltpu.SemaphoreType.DMA((NBUF,)),
        ],
    )
    def k(idx_hbm, tab_hbm, pos_hbm, out_hbm, idxv, posv, gbuf, gsem, osem):
        cid = lax.axis_index("c")
        sid = lax.axis_index("s")
        w = sid * NC + cid
        base = w * RPW

        def start_gather(g, b):
            pltpu.make_async_copy(
                tab_hbm.at[idxv.at[pl.ds(g * CH, CH)]],
                gbuf.at[b],
                gsem.at[b],
            ).start()

        def out_copy(g, b):
            return pltpu.make_async_copy(
                gbuf.at[b],
                out_hbm.at[pl.ds(base + g * CH, CH)],
                osem.at[b],
            )

        pltpu.sync_copy(idx_hbm.at[pl.ds(base, RPW)], idxv)
        start_gather(0, 0)
        start_gather(1, 1)
        pltpu.sync_copy(pos_hbm, posv.at[pl.ds(0, S)])
        pltpu.sync_copy(pos_hbm.at[pl.ds(0, PEXT)], posv.at[pl.ds(S, PEXT)])

        def step(g, b):
            b2 = (b + LOOK) % NBUF
            g2 = g + LOOK

            @pl.when(g2 < NCH)
            def _():
                @pl.when(g2 >= NBUF)
                def _():
                    out_copy(g2 - NBUF, b2).wait()

                start_gather(g2, b2)

            pltpu.make_async_copy(
                tab_hbm.at[idxv.at[pl.ds(g * CH, CH)]], gbuf.at[b], gsem.at[b]
            ).wait()
            p0 = lax.rem(g * CH, S)

            @pl.loop(0, CH)
            def _(r):
                pr = p0 + r
                for c in range(D // L):
                    sl = pl.ds(c * L, L)
                    gbuf[b, r, sl] = gbuf[b, r, sl] + posv[pr, sl]

            out_copy(g, b).start()

        @pl.loop(0, NCH // NBUF)
        def _(grp):
            for b in range(NBUF):
                step(grp * NBUF + b, b)

        for b in range(NBUF):
            out_copy(NCH - NBUF + b, b).wait()

    return k


def kernel(inputs, token_table, pos_table):
    B, S = inputs.shape
    V, D = token_table.shape
    N = B * S
    idx = inputs.reshape(N).astype(jnp.int32)
    run = _build_sc_kernel(N, V, D, S)
    out = run(idx, token_table, pos_table)
    return out.reshape(B, S, D)


# speedup vs baseline: 5.9960x; 2.3424x over previous
import functools

import jax
import jax.numpy as jnp
from jax import lax
from jax.experimental import pallas as pl
from jax.experimental.pallas import tpu as pltpu
from jax.experimental.pallas import tpu_sc as plsc


def _build_sc_kernel(N, V, D, S, B):
    info = plsc.get_sparse_core_info()
    NC, NS, L = info.num_cores, info.num_subcores, info.num_lanes
    NW = NC * NS
    RPW = N // NW
    CH = 64
    NCH = RPW // CH
    NBK = B // CH
    NBUF = 4
    LOOK = 2
    assert N % NW == 0 and RPW % CH == 0 and B % CH == 0 and D % L == 0
    assert CH % L == 0 and CH % 8 == 0 and NCH % NBUF == 0

    mesh = plsc.VectorSubcoreMesh(core_axis_name="c", subcore_axis_name="s")

    @functools.partial(
        pl.kernel,
        mesh=mesh,
        out_type=jax.ShapeDtypeStruct((N, D), jnp.float32),
        scratch_types=[
            pltpu.VMEM((RPW,), jnp.int32),
            pltpu.VMEM((S, D), jnp.float32),
            pltpu.VMEM((NBUF, CH, D), jnp.float32),
            pltpu.VMEM((NBUF, 1, CH), jnp.int32),
            pltpu.SemaphoreType.DMA((NBUF,)),
            pltpu.SemaphoreType.DMA((NBUF,)),
        ],
    )
    def k(tid_hbm, tab_hbm, pos_hbm, out_hbm, tidv, posv, gbuf, nlv, gsem, osem):
        cid = lax.axis_index("c")
        sid = lax.axis_index("s")
        w = sid * NC + cid
        t0 = w * NCH
        iota = lax.iota(jnp.int32, L) * S

        def start_gather(j, b):
            pltpu.make_async_copy(
                tab_hbm.at[tidv.at[pl.ds(j * CH, CH)]],
                gbuf.at[b],
                gsem.at[b],
            ).start()

        def out_copy(b):
            return pltpu.make_async_copy(
                gbuf.at[b],
                out_hbm.at[nlv.at[b, 0]],
                osem.at[b],
            )

        pltpu.sync_copy(tid_hbm.at[pl.ds(t0 * CH, RPW)], tidv)
        start_gather(0, 0)
        start_gather(1, 1)
        pltpu.sync_copy(pos_hbm, posv)

        def step(j, b):
            b2 = (b + LOOK) % NBUF
            j2 = j + LOOK

            @pl.when(j2 < NCH)
            def _():
                @pl.when(j2 >= NBUF)
                def _():
                    out_copy(b2).wait()

                start_gather(j2, b2)

            t = t0 + j
            s = t // NBK
            off = (t % NBK) * CH * S + s

            pv = [posv[s, pl.ds(c * L, L)] for c in range(D // L)]

            pltpu.make_async_copy(
                tab_hbm.at[tidv.at[pl.ds(j * CH, CH)]], gbuf.at[b], gsem.at[b]
            ).wait()

            for r in range(CH):
                for c in range(D // L):
                    sl = pl.ds(c * L, L)
                    gbuf[b, r, sl] = gbuf[b, r, sl] + pv[c]

            for c in range(CH // L):
                nlv[b, 0, pl.ds(c * L, L)] = iota + (off + c * L * S)

            out_copy(b).start()

        @pl.loop(0, NCH // NBUF)
        def _(grp):
            for b in range(NBUF):
                step(grp * NBUF + b, b)

        for b in range(NBUF):
            out_copy(b).wait()

    return k


def kernel(inputs, token_table, pos_table):
    B, S = inputs.shape
    V, D = token_table.shape
    N = B * S
    tid = inputs.T.reshape(N).astype(jnp.int32)
    run = _build_sc_kernel(N, V, D, S, B)
    out = run(tid, token_table, pos_table)
    return out.reshape(B, S, D)
